# baseline (speedup 1.0000x reference)
import jax
import jax.numpy as jnp
from jax.experimental import pallas as pl
from jax.experimental.pallas import tpu as pltpu

N = 1024
F_IN = 128
HID = 512
H = 8
DH = HID // H
L = 6
NODES_PER_SUB = 8
S = N // NODES_PER_SUB


def _gat_body(x_ref, adjb_ref, lw_ref, lb_ref, w_ref, asm_ref, adm_ref, admt_ref,
              node_ref, sub_ref, prob_ref):
    f32 = jnp.float32
    bf16 = jnp.bfloat16
    h0 = jnp.dot(x_ref[...], lw_ref[...], preferred_element_type=f32) + lb_ref[...]
    h = jnp.dot(h0, w_ref[...], preferred_element_type=f32)
    asrc = jnp.dot(h, asm_ref[...], preferred_element_type=f32)
    adst_col = jnp.dot(h, admt_ref[...], preferred_element_type=f32)
    adst_row = jax.lax.dot_general(adm_ref[...], h, (((1,), (1,)), ((), ())),
                                   preferred_element_type=f32)
    e1 = jnp.exp(asrc)
    e2 = jnp.exp(0.2 * asrc)
    g1 = jnp.exp(adst_col)
    g2 = jnp.exp(0.2 * adst_col)
    r8 = jax.lax.broadcasted_iota(jnp.int32, (H, HID), 0)
    c8 = jax.lax.broadcasted_iota(jnp.int32, (H, HID), 1)
    rep = jnp.where(c8 // DH == r8, 1.0, 0.0).astype(f32)
    v1 = (jnp.dot(g1, rep, preferred_element_type=f32) * h).astype(bf16)
    v2 = (jnp.dot(g2, rep, preferred_element_type=f32) * h).astype(bf16)
    g1b = g1.astype(bf16)
    g2b = g2.astype(bf16)
    adjb = adjb_ref[...]
    negs = (-asrc).astype(bf16)
    drow = adst_row.astype(bf16)
    zero = jnp.zeros((), bf16)
    for hd in range(H):
        msk = drow[hd:hd + 1, :] > negs[:, hd:hd + 1]
        p1 = jnp.where(msk, adjb, zero)
        p2 = adjb - p1
        w1 = jnp.concatenate([v1[:, hd * DH:(hd + 1) * DH], g1b[:, hd:hd + 1]], axis=1)
        w2 = jnp.concatenate([v2[:, hd * DH:(hd + 1) * DH], g2b[:, hd:hd + 1]], axis=1)
        u1 = jnp.dot(p1, w1, preferred_element_type=f32)
        u2 = jnp.dot(p2, w2, preferred_element_type=f32)
        c1 = e1[:, hd:hd + 1]
        c2 = e2[:, hd:hd + 1]
        ov = c1 * u1[:, :DH] + c2 * u2[:, :DH]
        rs = c1 * u1[:, DH:] + c2 * u2[:, DH:]
        o = ov / rs
        node_ref[:, hd * DH:(hd + 1) * DH] = jnp.where(o > 0, o, jnp.exp(o) - 1.0)
    r = jax.lax.broadcasted_iota(jnp.int32, (S, N), 0)
    c = jax.lax.broadcasted_iota(jnp.int32, (S, N), 1)
    pool = jnp.where(c // NODES_PER_SUB == r, 1.0 / NODES_PER_SUB, 0.0).astype(f32)
    sub_ref[...] = jnp.dot(pool, node_ref[...], preferred_element_type=f32)
    prob_ref[...] = jnp.ones((S, 1), f32)


def kernel(x, adj, lin_w, lin_b, gat_w, gat_a_src, gat_a_dst, cls_w, cls_b):
    f32 = jnp.float32
    w = gat_w[L - 1]
    a_src = gat_a_src[L - 1]
    a_dst = gat_a_dst[L - 1]
    eye = jnp.eye(H, dtype=f32)
    asm = (eye[:, :, None] * a_src[:, None, :]).reshape(H, HID).T
    adm = (eye[:, :, None] * a_dst[:, None, :]).reshape(H, HID)
    node, sub, prob = pl.pallas_call(
        _gat_body,
        out_shape=(
            jax.ShapeDtypeStruct((N, HID), f32),
            jax.ShapeDtypeStruct((S, HID), f32),
            jax.ShapeDtypeStruct((S, 1), f32),
        ),
    )(x, adj.astype(jnp.bfloat16), lin_w, lin_b.reshape(1, HID), w, asm, adm, adm.T)
    return (prob, node, sub)
